# Initial kernel scaffold; baseline (speedup 1.0000x reference)
import jax
import jax.numpy as jnp
from jax.experimental import pallas as pl


def kernel(normed_x, idx_last, k_global, v_global, Wq, Wk, Wv, Wproj):
    raise NotImplementedError("write your pallas kernel here")


# speedup vs baseline: 2.1652x; 2.1652x over previous
import functools

import jax
import jax.numpy as jnp
from jax import lax
from jax.experimental import pallas as pl
from jax.experimental.pallas import tpu as pltpu
from jax.experimental.pallas import tpu_sc as plsc

N = 50176
D = 96
HEADS = 4
HD = 24
HP = 32
DP = HEADS * HP
GS = 128
NG = N // GS
CG = 8
NCHUNK = NG // CG
NT = 8

NC = 2
NS = 16
NW = NC * NS
RPW = N // NW
CCH = 112
KCH = RPW // CCH

@functools.cache
def _sc_kernels():
    mesh = plsc.VectorSubcoreMesh(core_axis_name="c", subcore_axis_name="s")
    common = dict(
        mesh=mesh,
        compiler_params=pltpu.CompilerParams(use_tc_tiling_on_sc=False),
        out_type=jax.ShapeDtypeStruct((N, D), jnp.float32),
        scratch_types=[
            pltpu.VMEM((KCH, CCH), jnp.int32),
            pltpu.VMEM((CCH, D), jnp.float32),
            pltpu.SemaphoreType.DMA,
        ],
    )

    @functools.partial(pl.kernel, **common)
    def sc_gather(x_hbm, idx_hbm, out_hbm, idx_v, rows_v, sem):
        wid = lax.axis_index("s") * NC + lax.axis_index("c")
        base = wid * RPW
        pltpu.sync_copy(idx_hbm.at[wid], idx_v)
        for j in range(KCH):
            pltpu.async_copy(x_hbm.at[idx_v.at[j]], rows_v, sem).wait()
            pltpu.sync_copy(rows_v, out_hbm.at[pl.ds(base + j * CCH, CCH)])

    @functools.partial(pl.kernel, **common)
    def sc_scatter(o_hbm, idx_hbm, out_hbm, idx_v, rows_v, sem):
        wid = lax.axis_index("s") * NC + lax.axis_index("c")
        base = wid * RPW
        pltpu.sync_copy(idx_hbm.at[wid], idx_v)
        for j in range(KCH):
            pltpu.sync_copy(o_hbm.at[pl.ds(base + j * CCH, CCH)], rows_v)
            pltpu.async_copy(rows_v, out_hbm.at[idx_v.at[j]], sem).wait()

    return sc_gather, sc_scatter


def _attn_body(xa_ref, xb_ref, wq_ref, wk_ref, wv_ref, wp_ref, kg_ref,
               vg_ref, out_ref):
    f32 = jnp.float32
    xa = xa_ref[...]
    xb = xb_ref[...]
    wq = wq_ref[...]
    wk = wk_ref[...]
    wv = wv_ref[...]
    wp = wp_ref[...]
    kg = kg_ref[...]
    vg = vg_ref[...]

    def dot(a, b, dn):
        return lax.dot_general(a, b, dn, preferred_element_type=f32)

    q = dot(xa, wq, (((1,), (1,)), ((), ())))
    xc = jnp.concatenate([xa, xb], axis=0)
    kf = dot(xc, wk, (((1,), (1,)), ((), ())))
    vf = dot(xc, wv, (((1,), (1,)), ((), ())))

    scale = HD ** -0.5
    acc = jnp.zeros((CG, GS, D), f32)
    for h in range(HEADS):
        sl = slice(h * HP, (h + 1) * HP)
        qh = q[:, sl].reshape(CG, GS, HP)
        kh = jnp.concatenate(
            [kf[:CG * GS, sl].reshape(CG, GS, HP),
             kf[GS:, sl].reshape(CG, GS, HP)], axis=1)
        vh = jnp.concatenate(
            [vf[:CG * GS, sl].reshape(CG, GS, HP),
             vf[GS:, sl].reshape(CG, GS, HP)], axis=1)
        s = dot(qh, kh, (((2,), (2,)), ((0,), (0,)))) * scale
        m = jnp.max(s, axis=-1, keepdims=True)
        p = jnp.exp(s - m)
        l = jnp.sum(p, axis=-1, keepdims=True)
        o1 = dot(p, vh, (((2,), (1,)), ((0,), (0,)))) / l
        kgh = kg[h * NT:(h + 1) * NT, :]
        vgh = vg[h * NT:(h + 1) * NT, :]
        sg = dot(qh, kgh, (((2,), (1,)), ((), ()))) * scale
        mg = jnp.max(sg, axis=-1, keepdims=True)
        pg = jnp.exp(sg - mg)
        lg = jnp.sum(pg, axis=-1, keepdims=True)
        o2 = dot(pg, vgh, (((2,), (0,)), ((), ()))) / lg
        acc = acc + dot(o1 + o2, wp[:, sl], (((2,), (1,)), ((), ())))
    out_ref[...] = acc.reshape(CG * GS, D)


_attn = pl.pallas_call(
    _attn_body,
    grid=(NCHUNK,),
    in_specs=[
        pl.BlockSpec((CG * GS, D), lambda c: (c, 0)),
        pl.BlockSpec((GS, D), lambda c: (jnp.minimum(CG * c + CG, NG - 1), 0)),
        pl.BlockSpec((DP, D), lambda c: (0, 0)),
        pl.BlockSpec((DP, D), lambda c: (0, 0)),
        pl.BlockSpec((DP, D), lambda c: (0, 0)),
        pl.BlockSpec((D, DP), lambda c: (0, 0)),
        pl.BlockSpec((HEADS * NT, HP), lambda c: (0, 0)),
        pl.BlockSpec((HEADS * NT, HP), lambda c: (0, 0)),
    ],
    out_specs=pl.BlockSpec((CG * GS, D), lambda c: (c, 0)),
    out_shape=jax.ShapeDtypeStruct((N, D), jnp.float32),
)


def _pad_heads_rows(w):
    return jnp.pad(w.reshape(HEADS, HD, D),
                   ((0, 0), (0, HP - HD), (0, 0))).reshape(DP, D)


def kernel(normed_x, idx_last, k_global, v_global, Wq, Wk, Wv, Wproj):
    x = normed_x[0]
    perm = idx_last[0, :, 0].astype(jnp.int32)
    idx3 = perm.reshape(NW, KCH, CCH)

    wq = _pad_heads_rows(Wq)
    wk = _pad_heads_rows(Wk)
    wv = _pad_heads_rows(Wv)
    wp = jnp.pad(Wproj.reshape(D, HEADS, HD),
                 ((0, 0), (0, 0), (0, HP - HD))).reshape(D, DP)
    kg = jnp.pad(k_global, ((0, 0), (0, 0), (0, HP - HD))).reshape(
        HEADS * NT, HP)
    vg = jnp.pad(v_global, ((0, 0), (0, 0), (0, HP - HD))).reshape(
        HEADS * NT, HP)

    sc_gather, sc_scatter = _sc_kernels()
    x_perm = sc_gather(x, idx3)
    o = _attn(x_perm, x_perm, wq, wk, wv, wp, kg, vg)
    y = sc_scatter(o, idx3)
    return y[None]
